# Initial kernel scaffold; baseline (speedup 1.0000x reference)
import jax
import jax.numpy as jnp
from jax.experimental import pallas as pl


def kernel(x, table):
    raise NotImplementedError("write your pallas kernel here")


# speedup vs baseline: 4.5770x; 4.5770x over previous
import functools

import jax
import jax.numpy as jnp
from jax import lax
from jax.experimental import pallas as pl
from jax.experimental.pallas import tpu as pltpu
from jax.experimental.pallas import tpu_sc as plsc

BATCH = 16384
HIST = 200
D = 64
B = BATCH * HIST
NW = 32
RPX = 128
K = 4
G = K * RPX
B_PER_W = B // NW
XROWS_PER_W = B_PER_W // RPX
NCH = B_PER_W // G

_mesh = plsc.VectorSubcoreMesh(core_axis_name="c", subcore_axis_name="s")


@functools.partial(
    pl.kernel,
    mesh=_mesh,
    compiler_params=pltpu.CompilerParams(use_tc_tiling_on_sc=False),
    out_type=jax.ShapeDtypeStruct((B, D), jnp.float32),
    scratch_types=[
        pltpu.VMEM((K, RPX), jnp.int32),
        pltpu.VMEM((G, D), jnp.float32),
        pltpu.SemaphoreType.DMA,
    ],
)
def _gather_kernel(x_hbm, table_hbm, out_hbm, idx_v, rows_v, gsem):
    wid = lax.axis_index("s") * 2 + lax.axis_index("c")
    row0 = wid * XROWS_PER_W

    def body(c, carry):
        krow = row0 + c * K
        pltpu.sync_copy(x_hbm.at[pl.ds(krow, K)], idx_v)
        descs = [
            pltpu.async_copy(
                table_hbm.at[idx_v.at[j]],
                rows_v.at[pl.ds(j * RPX, RPX)],
                gsem,
            )
            for j in range(K)
        ]
        for d in descs:
            d.wait()
        pltpu.sync_copy(rows_v, out_hbm.at[pl.ds(krow * RPX, G)])
        return carry

    lax.fori_loop(0, NCH, body, 0)


def kernel(x, table):
    x2 = x.astype(jnp.int32).reshape(B // RPX, RPX)
    out = _gather_kernel(x2, table)
    return out.reshape(BATCH, HIST, D)


# speedup vs baseline: 4.7306x; 1.0336x over previous
import functools

import jax
import jax.numpy as jnp
from jax import lax
from jax.experimental import pallas as pl
from jax.experimental.pallas import tpu as pltpu
from jax.experimental.pallas import tpu_sc as plsc

BATCH = 16384
HIST = 200
D = 64
B = BATCH * HIST
NW = 32
RPX = 128
K = 4
G = K * RPX
B_PER_W = B // NW
XROWS_PER_W = B_PER_W // RPX
NCH = B_PER_W // G
NP = NCH // 2

_mesh = plsc.VectorSubcoreMesh(core_axis_name="c", subcore_axis_name="s")


@functools.partial(
    pl.kernel,
    mesh=_mesh,
    compiler_params=pltpu.CompilerParams(use_tc_tiling_on_sc=False),
    out_type=jax.ShapeDtypeStruct((B, D), jnp.float32),
    scratch_types=[
        pltpu.VMEM((2 * K, RPX), jnp.int32),
        pltpu.VMEM((G, D), jnp.float32),
        pltpu.VMEM((G, D), jnp.float32),
        pltpu.SemaphoreType.DMA,
        pltpu.SemaphoreType.DMA,
        pltpu.SemaphoreType.DMA,
    ],
)
def _gather_kernel(x_hbm, table_hbm, out_hbm, idx_v, rows0, rows1,
                   gsem0, gsem1, ssem):
    wid = lax.axis_index("s") * 2 + lax.axis_index("c")
    row0 = wid * XROWS_PER_W

    def load_idx(krow, lo, n):
        pltpu.sync_copy(x_hbm.at[pl.ds(row0 + krow, n)],
                        idx_v.at[pl.ds(lo, n)])

    def fire_gathers(lo, rows, sem):
        for j in range(K):
            pltpu.async_copy(table_hbm.at[idx_v.at[lo + j]],
                             rows.at[pl.ds(j * RPX, RPX)], sem)

    def drain_gathers(rows, sem):
        for j in range(K):
            pltpu.make_async_copy(table_hbm.at[idx_v.at[j]],
                                  rows.at[pl.ds(j * RPX, RPX)], sem).wait()

    def fire_store(c, rows):
        pltpu.async_copy(rows, out_hbm.at[pl.ds((row0 + c * K) * RPX, G)],
                         ssem)

    def wait_store(c, rows):
        pltpu.make_async_copy(rows,
                              out_hbm.at[pl.ds((row0 + c * K) * RPX, G)],
                              ssem).wait()

    load_idx(0, 0, K)
    fire_gathers(0, rows0, gsem0)

    def body(p, carry):
        c = 2 * p
        drain_gathers(rows0, gsem0)
        fire_store(c, rows0)
        load_idx((c + 1) * K, 0, 2 * K)
        fire_gathers(0, rows1, gsem1)
        wait_store(c, rows0)
        fire_gathers(K, rows0, gsem0)
        drain_gathers(rows1, gsem1)
        fire_store(c + 1, rows1)
        wait_store(c + 1, rows1)
        return carry

    lax.fori_loop(0, NP - 1, body, 0)

    drain_gathers(rows0, gsem0)
    fire_store(NCH - 2, rows0)
    load_idx((NCH - 1) * K, 0, K)
    fire_gathers(0, rows1, gsem1)
    wait_store(NCH - 2, rows0)
    drain_gathers(rows1, gsem1)
    fire_store(NCH - 1, rows1)
    wait_store(NCH - 1, rows1)


def kernel(x, table):
    x2 = x.astype(jnp.int32).reshape(B // RPX, RPX)
    out = _gather_kernel(x2, table)
    return out.reshape(BATCH, HIST, D)


# speedup vs baseline: 6.1663x; 1.3035x over previous
import functools

import jax
import jax.numpy as jnp
from jax import lax
from jax.experimental import pallas as pl
from jax.experimental.pallas import tpu as pltpu
from jax.experimental.pallas import tpu_sc as plsc

BATCH = 16384
HIST = 200
D = 64
VOCAB = 2048
L = 16
NDG = 4
NBG = 8
DG = D // NDG
BG = BATCH // NBG
HB = 8
NHB = HIST // HB
NIV = BG // L
TILE_W = 1024
STG_W = DG * BG
HALF_STG = STG_W // 2

_mesh = plsc.VectorSubcoreMesh(core_axis_name="c", subcore_axis_name="s")


@functools.partial(
    pl.kernel,
    mesh=_mesh,
    compiler_params=pltpu.CompilerParams(
        use_tc_tiling_on_sc=False, needs_layout_passes=False),
    out_type=jax.ShapeDtypeStruct((HIST * D * BATCH,), jnp.float32),
    scratch_types=[
        pltpu.VMEM((DG * VOCAB,), jnp.float32),
        pltpu.VMEM((HB, BG), jnp.int32),
        pltpu.VMEM((STG_W,), jnp.float32),
        pltpu.SemaphoreType.DMA,
    ],
)
def _lookup_kernel(xt_hbm, tt_hbm, out_hbm, tbl_v, xblk_v, stg_v, sem):
    wid = lax.axis_index("s") * 2 + lax.axis_index("c")
    dg = wid % NDG
    bg = wid // NDG
    d0 = dg * DG
    b0 = bg * BG

    for dl in range(DG):
        pltpu.sync_copy(tt_hbm.at[d0 + dl],
                        tbl_v.at[pl.ds(dl * VOCAB, VOCAB)])

    def hblock(hb, carry):
        pltpu.sync_copy(xt_hbm.at[pl.ds(hb * HB, HB), pl.ds(b0, BG)], xblk_v)
        for hl in range(HB):
            h = hb * HB + hl

            def inner(i, c2):
                xv = xblk_v[hl, pl.ds(i * L, L)]
                base = (i // 8) * TILE_W + (i % 8) * L
                for dl in range(DG):
                    off = base + (dl // 8) * HALF_STG + (dl % 8) * 128
                    stg_v[pl.ds(off, L)] = (
                        plsc.load_gather(tbl_v, [xv + dl * VOCAB]))
                return c2

            lax.fori_loop(0, NIV, inner, 0)
            for j in range(2):
                dst = ((h * 8 + 2 * dg + j) * 128 + 16 * bg) * TILE_W
                pltpu.sync_copy(stg_v.at[pl.ds(j * HALF_STG, HALF_STG)],
                                out_hbm.at[pl.ds(dst, HALF_STG)])
        return carry

    lax.fori_loop(0, NHB, hblock, 0)


def kernel(x, table):
    xt = x.astype(jnp.int32).T
    tt = table.T
    out_f = _lookup_kernel(xt, tt)
    out5 = out_f.reshape(HIST, 8, 128, 8, 128)
    return out5.transpose(2, 4, 0, 1, 3).reshape(BATCH, HIST, D)


# speedup vs baseline: 19.3291x; 3.1346x over previous
import functools

import jax
import jax.numpy as jnp
from jax import lax
from jax.experimental import pallas as pl
from jax.experimental.pallas import tpu as pltpu
from jax.experimental.pallas import tpu_sc as plsc

BATCH = 16384
HIST = 200
D = 64
VOCAB = 2048
L = 16
NDG = 4
NBG = 8
DG = D // NDG
BG = BATCH // NBG
HB = 8
NHB = HIST // HB
NIV = BG // L
TILE_W = 1024
STG_W = DG * BG
HALF_STG = STG_W // 2

_mesh = plsc.VectorSubcoreMesh(core_axis_name="c", subcore_axis_name="s")


@functools.partial(
    pl.kernel,
    mesh=_mesh,
    compiler_params=pltpu.CompilerParams(
        use_tc_tiling_on_sc=False, needs_layout_passes=False),
    out_type=jax.ShapeDtypeStruct((HIST * D * BATCH,), jnp.float32),
    scratch_types=[
        pltpu.VMEM((DG * VOCAB,), jnp.float32),
        pltpu.VMEM((HB, BG), jnp.int32),
        pltpu.VMEM((STG_W,), jnp.float32),
        pltpu.SemaphoreType.DMA,
    ],
)
def _lookup_kernel(xt_hbm, tt_hbm, out_hbm, tbl_v, xblk_v, stg_v, sem):
    wid = lax.axis_index("s") * 2 + lax.axis_index("c")
    dg = wid % NDG
    bg = wid // NDG
    d0 = dg * DG
    b0 = bg * BG

    for dl in range(DG):
        pltpu.sync_copy(tt_hbm.at[d0 + dl],
                        tbl_v.at[pl.ds(dl * VOCAB, VOCAB)])

    def hblock(hb, carry):
        pltpu.sync_copy(xt_hbm.at[pl.ds(hb * HB, HB), pl.ds(b0, BG)], xblk_v)
        for hl in range(HB):
            h = hb * HB + hl

            @plsc.parallel_loop(0, NIV, unroll=4)
            def inner(i):
                xv = xblk_v[hl, pl.ds(i * L, L)]
                base = (i // 8) * TILE_W + (i % 8) * L
                for dl in range(DG):
                    off = base + (dl // 8) * HALF_STG + (dl % 8) * 128
                    stg_v[pl.ds(off, L)] = plsc.load_gather(
                        tbl_v.at[pl.ds(dl * VOCAB, VOCAB)], [xv])
            for j in range(2):
                dst = ((h * 8 + 2 * dg + j) * 128 + 16 * bg) * TILE_W
                pltpu.sync_copy(stg_v.at[pl.ds(j * HALF_STG, HALF_STG)],
                                out_hbm.at[pl.ds(dst, HALF_STG)])
        return carry

    lax.fori_loop(0, NHB, hblock, 0)


def kernel(x, table):
    xt = x.astype(jnp.int32).T
    tt = table.T
    out_f = _lookup_kernel(xt, tt)
    out5 = out_f.reshape(HIST, 8, 128, 8, 128)
    return out5.transpose(2, 4, 0, 1, 3).reshape(BATCH, HIST, D)


# speedup vs baseline: 28.5568x; 1.4774x over previous
import functools

import jax
import jax.numpy as jnp
from jax import lax
from jax.experimental import pallas as pl
from jax.experimental.pallas import tpu as pltpu
from jax.experimental.pallas import tpu_sc as plsc

BATCH = 16384
HIST = 200
D = 64
VOCAB = 2048
L = 16
NDG = 4
NBG = 8
DG = D // NDG
BG = BATCH // NBG
HB = 8
NHB = HIST // HB
NIV = BG // L
TILE_W = 1024
STG_W = DG * BG
HALF_STG = STG_W // 2

_mesh = plsc.VectorSubcoreMesh(core_axis_name="c", subcore_axis_name="s")


@functools.partial(
    pl.kernel,
    mesh=_mesh,
    compiler_params=pltpu.CompilerParams(
        use_tc_tiling_on_sc=False, needs_layout_passes=False),
    out_type=jax.ShapeDtypeStruct((HIST * D * BATCH,), jnp.float32),
    scratch_types=[
        pltpu.VMEM((DG * VOCAB,), jnp.float32),
        pltpu.VMEM((HB, BG), jnp.int32),
        pltpu.VMEM((STG_W,), jnp.float32),
        pltpu.VMEM((STG_W,), jnp.float32),
        pltpu.SemaphoreType.DMA,
        pltpu.SemaphoreType.DMA,
    ],
)
def _lookup_kernel(xt_hbm, tt_hbm, out_hbm, tbl_v, xblk_v, stg_a, stg_b,
                   sem_a, sem_b):
    wid = lax.axis_index("s") * 2 + lax.axis_index("c")
    dg = wid % NDG
    bg = wid // NDG
    d0 = dg * DG
    b0 = bg * BG

    for dl in range(DG):
        pltpu.sync_copy(tt_hbm.at[d0 + dl],
                        tbl_v.at[pl.ds(dl * VOCAB, VOCAB)])

    def store_descs(h, stg, sem):
        return [
            pltpu.make_async_copy(
                stg.at[pl.ds(j * HALF_STG, HALF_STG)],
                out_hbm.at[pl.ds(
                    ((h * 8 + 2 * dg + j) * 128 + 16 * bg) * TILE_W,
                    HALF_STG)],
                sem)
            for j in range(2)
        ]

    def hblock(hb, carry):
        pltpu.sync_copy(xt_hbm.at[pl.ds(hb * HB, HB), pl.ds(b0, BG)], xblk_v)
        for hl in range(HB):
            h = hb * HB + hl
            stg, sem = (stg_a, sem_a) if hl % 2 == 0 else (stg_b, sem_b)

            def drain():
                for d_ in store_descs(h, stg, sem):
                    d_.wait()

            if hl >= 2:
                drain()
            else:
                @pl.when(hb > 0)
                def _():
                    drain()

            @plsc.parallel_loop(0, NIV, unroll=4)
            def inner(i):
                xv = xblk_v[hl, pl.ds(i * L, L)]
                base = (i // 8) * TILE_W + (i % 8) * L
                for dl in range(DG):
                    off = base + (dl // 8) * HALF_STG + (dl % 8) * 128
                    stg_v_slice = tbl_v.at[pl.ds(dl * VOCAB, VOCAB)]
                    stg[pl.ds(off, L)] = plsc.load_gather(stg_v_slice, [xv])

            for d_ in store_descs(h, stg, sem):
                d_.start()
        return carry

    lax.fori_loop(0, NHB, hblock, 0)

    for d_ in store_descs(HIST - 2, stg_a, sem_a):
        d_.wait()
    for d_ in store_descs(HIST - 1, stg_b, sem_b):
        d_.wait()


def kernel(x, table):
    xt = x.astype(jnp.int32).T
    tt = table.T
    out_f = _lookup_kernel(xt, tt)
    out5 = out_f.reshape(HIST, 8, 128, 8, 128)
    return out5.transpose(2, 4, 0, 1, 3).reshape(BATCH, HIST, D)


# speedup vs baseline: 30.1204x; 1.0548x over previous
import functools

import jax
import jax.numpy as jnp
from jax import lax
from jax.experimental import pallas as pl
from jax.experimental.pallas import tpu as pltpu
from jax.experimental.pallas import tpu_sc as plsc

BATCH = 16384
HIST = 200
D = 64
VOCAB = 2048
L = 16
NDG = 4
NBG = 8
DG = D // NDG
BG = BATCH // NBG
HB = 8
NHB = HIST // HB
NIV = BG // L
TILE_W = 1024
XBLK_W = HB * BG
STG_W = DG * BG
HALF_STG = STG_W // 2

_mesh = plsc.VectorSubcoreMesh(core_axis_name="c", subcore_axis_name="s")


@functools.partial(
    pl.kernel,
    mesh=_mesh,
    compiler_params=pltpu.CompilerParams(
        use_tc_tiling_on_sc=False, needs_layout_passes=False),
    out_type=jax.ShapeDtypeStruct((HIST * D * BATCH,), jnp.float32),
    scratch_types=[
        pltpu.VMEM((DG * VOCAB,), jnp.float32),
        pltpu.VMEM((XBLK_W,), jnp.int32),
        pltpu.VMEM((STG_W,), jnp.float32),
        pltpu.VMEM((STG_W,), jnp.float32),
        pltpu.SemaphoreType.DMA,
        pltpu.SemaphoreType.DMA,
    ],
)
def _lookup_kernel(x4_hbm, tt_hbm, out_hbm, tbl_v, xblk_v, stg_a, stg_b,
                   sem_a, sem_b):
    wid = lax.axis_index("s") * 2 + lax.axis_index("c")
    dg = wid % NDG
    bg = wid // NDG
    d0 = dg * DG
    b0 = bg * BG

    pltpu.sync_copy(tt_hbm.at[pl.ds(d0 * VOCAB, DG * VOCAB)], tbl_v)

    def store_descs(h, stg, sem):
        return [
            pltpu.make_async_copy(
                stg.at[pl.ds(j * HALF_STG, HALF_STG)],
                out_hbm.at[pl.ds(
                    ((h * 8 + 2 * dg + j) * 128 + 16 * bg) * TILE_W,
                    HALF_STG)],
                sem)
            for j in range(2)
        ]

    def hblock(hb, carry):
        pltpu.sync_copy(
            x4_hbm.at[pl.ds((hb * 128 + 16 * bg) * TILE_W, XBLK_W)], xblk_v)
        for hl in range(HB):
            h = hb * HB + hl
            stg, sem = (stg_a, sem_a) if hl % 2 == 0 else (stg_b, sem_b)

            def drain():
                for d_ in store_descs(h, stg, sem):
                    d_.wait()

            if hl >= 2:
                drain()
            else:
                @pl.when(hb > 0)
                def _():
                    drain()

            @plsc.parallel_loop(0, NIV, unroll=4)
            def inner(i):
                base = (i // 8) * TILE_W + (i % 8) * L
                xv = xblk_v[pl.ds(base + hl * 128, L)]
                for dl in range(DG):
                    off = base + (dl // 8) * HALF_STG + (dl % 8) * 128
                    tbl_slice = tbl_v.at[pl.ds(dl * VOCAB, VOCAB)]
                    stg[pl.ds(off, L)] = plsc.load_gather(tbl_slice, [xv])

            for d_ in store_descs(h, stg, sem):
                d_.start()
        return carry

    lax.fori_loop(0, NHB, hblock, 0)

    for d_ in store_descs(HIST - 2, stg_a, sem_a):
        d_.wait()
    for d_ in store_descs(HIST - 1, stg_b, sem_b):
        d_.wait()


def kernel(x, table):
    x4 = (x.astype(jnp.int32)
          .reshape(128, 128, NHB, HB)
          .transpose(2, 0, 3, 1)
          .reshape(-1))
    tt = table.T.reshape(-1)
    out_f = _lookup_kernel(x4, tt)
    out5 = out_f.reshape(HIST, 8, 128, 8, 128)
    return out5.transpose(2, 4, 0, 1, 3).reshape(BATCH, HIST, D)
